# speedup vs baseline: 1.4291x; 1.4291x over previous
import functools

import jax
import jax.numpy as jnp
from jax import lax
from jax.experimental import pallas as pl
from jax.experimental.pallas import tpu as pltpu
from jax.experimental.pallas import tpu_sc as plsc

_NUM_CORES = 2
_NUM_SUBCORES = 16
_NUM_WORKERS = _NUM_CORES * _NUM_SUBCORES

_CHUNK = 16


def _dual_gather(n_tokens, from_dim, to_dim):
  n_per_w = n_tokens // _NUM_WORKERS
  n_chunks = n_per_w // _CHUNK

  mesh = plsc.VectorSubcoreMesh(
      core_axis_name="c", subcore_axis_name="s",
      num_cores=_NUM_CORES, num_subcores=_NUM_SUBCORES)

  @functools.partial(
      pl.kernel,
      out_type=(
          jax.ShapeDtypeStruct((n_tokens, from_dim), jnp.float32),
          jax.ShapeDtypeStruct((n_tokens, to_dim), jnp.float32),
      ),
      mesh=mesh,
      scratch_types=[
          pltpu.VMEM((n_chunks, _CHUNK), jnp.int32),
          pltpu.VMEM((_CHUNK, from_dim), jnp.float32),
          pltpu.VMEM((_CHUNK, to_dim), jnp.float32),
          pltpu.SemaphoreType.DMA,
      ],
  )
  def k(tok_hbm, from_hbm, to_hbm, out_from_hbm, out_to_hbm,
        idx_v, fbuf, tbuf, sem):
    wid = lax.axis_index("s") * _NUM_CORES + lax.axis_index("c")
    chunk_row = wid * n_chunks
    pltpu.sync_copy(tok_hbm.at[pl.ds(chunk_row, n_chunks)], idx_v)

    def body(j, carry):
      tok_base = (chunk_row + j) * _CHUNK
      pltpu.async_copy(from_hbm.at[idx_v.at[j]], fbuf, sem).wait()
      pltpu.sync_copy(fbuf, out_from_hbm.at[pl.ds(tok_base, _CHUNK)])
      pltpu.async_copy(to_hbm.at[idx_v.at[j]], tbuf, sem).wait()
      pltpu.sync_copy(tbuf, out_to_hbm.at[pl.ds(tok_base, _CHUNK)])
      return carry

    lax.fori_loop(0, n_chunks, body, 0)

  return k


def kernel(t5_tokens, from_table, to_table):
  batch, seq = t5_tokens.shape
  n_tokens = batch * seq
  from_dim = from_table.shape[1]
  to_dim = to_table.shape[1]

  tokens2d = t5_tokens.reshape(n_tokens // _CHUNK, _CHUNK)
  gather = _dual_gather(n_tokens, from_dim, to_dim)
  out_from, out_to = gather(tokens2d, from_table, to_table)
  return (out_from.reshape(batch, seq, from_dim),
          out_to.reshape(batch, seq, to_dim))


# speedup vs baseline: 1.8524x; 1.2962x over previous
import functools

import jax
import jax.numpy as jnp
from jax import lax
from jax.experimental import pallas as pl
from jax.experimental.pallas import tpu as pltpu
from jax.experimental.pallas import tpu_sc as plsc

_NUM_CORES = 2
_NUM_SUBCORES = 16
_NUM_WORKERS = _NUM_CORES * _NUM_SUBCORES

_CHUNK = 16


def _dual_gather(n_tokens, from_dim, to_dim):
  n_per_w = n_tokens // _NUM_WORKERS
  n_chunks = n_per_w // _CHUNK

  mesh = plsc.VectorSubcoreMesh(
      core_axis_name="c", subcore_axis_name="s",
      num_cores=_NUM_CORES, num_subcores=_NUM_SUBCORES)

  @functools.partial(
      pl.kernel,
      out_type=(
          jax.ShapeDtypeStruct((n_tokens, from_dim), jnp.float32),
          jax.ShapeDtypeStruct((n_tokens, to_dim), jnp.float32),
      ),
      mesh=mesh,
      scratch_types=[
          pltpu.VMEM((n_chunks, _CHUNK), jnp.int32),
          pltpu.VMEM((2, _CHUNK, from_dim), jnp.float32),
          pltpu.VMEM((2, _CHUNK, to_dim), jnp.float32),
      ] + [pltpu.SemaphoreType.DMA] * 8,
  )
  def k(tok_hbm, from_hbm, to_hbm, out_from_hbm, out_to_hbm,
        idx_v, fbuf, tbuf, gf0, gf1, gt0, gt1, sf0, sf1, st0, st1):
    wid = lax.axis_index("s") * _NUM_CORES + lax.axis_index("c")
    chunk_row = wid * n_chunks
    pltpu.sync_copy(tok_hbm.at[pl.ds(chunk_row, n_chunks)], idx_v)

    slots = ((fbuf.at[0], tbuf.at[0], gf0, gt0, sf0, st0),
             (fbuf.at[1], tbuf.at[1], gf1, gt1, sf1, st1))

    def gather_issue(j, fb, tb, gf, gt):
      pltpu.async_copy(from_hbm.at[idx_v.at[j]], fb, gf)
      pltpu.async_copy(to_hbm.at[idx_v.at[j]], tb, gt)

    def gather_wait(fb, tb, gf, gt):
      pltpu.make_async_copy(out_from_hbm.at[pl.ds(0, _CHUNK)], fb, gf).wait()
      pltpu.make_async_copy(out_to_hbm.at[pl.ds(0, _CHUNK)], tb, gt).wait()

    def store_issue(j, fb, tb, sf, st):
      base = (chunk_row + j) * _CHUNK
      pltpu.async_copy(fb, out_from_hbm.at[pl.ds(base, _CHUNK)], sf)
      pltpu.async_copy(tb, out_to_hbm.at[pl.ds(base, _CHUNK)], st)

    def store_wait(fb, tb, sf, st):
      pltpu.make_async_copy(fb, out_from_hbm.at[pl.ds(0, _CHUNK)], sf).wait()
      pltpu.make_async_copy(tb, out_to_hbm.at[pl.ds(0, _CHUNK)], st).wait()

    for b in range(2):
      fb, tb, gf, gt, _, _ = slots[b]
      gather_issue(b, fb, tb, gf, gt)

    def body(i, carry):
      for b in range(2):
        j = 2 * i + b
        fb, tb, gf, gt, sf, st = slots[b]
        gather_wait(fb, tb, gf, gt)
        store_issue(j, fb, tb, sf, st)

        @pl.when(j + 2 < n_chunks)
        def _():
          store_wait(fb, tb, sf, st)
          gather_issue(j + 2, fb, tb, gf, gt)

      return carry

    lax.fori_loop(0, n_chunks // 2, body, 0)

    for b in range(2):
      fb, tb, _, _, sf, st = slots[b]
      store_wait(fb, tb, sf, st)

  return k


def kernel(t5_tokens, from_table, to_table):
  batch, seq = t5_tokens.shape
  n_tokens = batch * seq
  from_dim = from_table.shape[1]
  to_dim = to_table.shape[1]

  tokens2d = t5_tokens.reshape(n_tokens // _CHUNK, _CHUNK)
  gather = _dual_gather(n_tokens, from_dim, to_dim)
  out_from, out_to = gather(tokens2d, from_table, to_table)
  return (out_from.reshape(batch, seq, from_dim),
          out_to.reshape(batch, seq, to_dim))


# speedup vs baseline: 1.8610x; 1.0047x over previous
import functools

import jax
import jax.numpy as jnp
from jax import lax
from jax.experimental import pallas as pl
from jax.experimental.pallas import tpu as pltpu
from jax.experimental.pallas import tpu_sc as plsc

_NUM_CORES = 2
_NUM_SUBCORES = 16
_NUM_WORKERS = _NUM_CORES * _NUM_SUBCORES

_CHUNK = 8
_NBUF = 4


def _dual_gather(n_tokens, from_dim, to_dim):
  n_per_w = n_tokens // _NUM_WORKERS
  n_chunks = n_per_w // _CHUNK

  mesh = plsc.VectorSubcoreMesh(
      core_axis_name="c", subcore_axis_name="s",
      num_cores=_NUM_CORES, num_subcores=_NUM_SUBCORES)

  @functools.partial(
      pl.kernel,
      out_type=(
          jax.ShapeDtypeStruct((n_tokens, from_dim), jnp.float32),
          jax.ShapeDtypeStruct((n_tokens, to_dim), jnp.float32),
      ),
      mesh=mesh,
      scratch_types=[
          pltpu.VMEM((n_chunks, _CHUNK), jnp.int32),
          pltpu.VMEM((_NBUF, _CHUNK, from_dim), jnp.float32),
          pltpu.VMEM((_NBUF, _CHUNK, to_dim), jnp.float32),
          [pltpu.SemaphoreType.DMA] * _NBUF,
          [pltpu.SemaphoreType.DMA] * _NBUF,
          [pltpu.SemaphoreType.DMA] * _NBUF,
          [pltpu.SemaphoreType.DMA] * _NBUF,
      ],
  )
  def k(tok_hbm, from_hbm, to_hbm, out_from_hbm, out_to_hbm,
        idx_v, fbuf, tbuf, gf, gt, sf, st):
    wid = lax.axis_index("s") * _NUM_CORES + lax.axis_index("c")
    chunk_row = wid * n_chunks
    pltpu.sync_copy(tok_hbm.at[pl.ds(chunk_row, n_chunks)], idx_v)

    slots = tuple(
        (fbuf.at[b], tbuf.at[b], gf[b], gt[b], sf[b], st[b])
        for b in range(_NBUF))

    def gather_issue(j, fb, tb, gfs, gts):
      pltpu.async_copy(from_hbm.at[idx_v.at[j]], fb, gfs)
      pltpu.async_copy(to_hbm.at[idx_v.at[j]], tb, gts)

    def gather_wait(fb, tb, gfs, gts):
      pltpu.make_async_copy(out_from_hbm.at[pl.ds(0, _CHUNK)], fb, gfs).wait()
      pltpu.make_async_copy(out_to_hbm.at[pl.ds(0, _CHUNK)], tb, gts).wait()

    def store_issue(j, fb, tb, sfs, sts):
      base = (chunk_row + j) * _CHUNK
      pltpu.async_copy(fb, out_from_hbm.at[pl.ds(base, _CHUNK)], sfs)
      pltpu.async_copy(tb, out_to_hbm.at[pl.ds(base, _CHUNK)], sts)

    def store_wait(fb, tb, sfs, sts):
      pltpu.make_async_copy(fb, out_from_hbm.at[pl.ds(0, _CHUNK)], sfs).wait()
      pltpu.make_async_copy(tb, out_to_hbm.at[pl.ds(0, _CHUNK)], sts).wait()

    for b in range(_NBUF):
      fb, tb, gfs, gts, _, _ = slots[b]
      gather_issue(b, fb, tb, gfs, gts)

    def body(i, carry):
      for b in range(_NBUF):
        j = _NBUF * i + b
        fb, tb, gfs, gts, sfs, sts = slots[b]
        gather_wait(fb, tb, gfs, gts)
        store_issue(j, fb, tb, sfs, sts)

        @pl.when(j + _NBUF < n_chunks)
        def _():
          store_wait(fb, tb, sfs, sts)
          gather_issue(j + _NBUF, fb, tb, gfs, gts)

      return carry

    lax.fori_loop(0, n_chunks // _NBUF, body, 0)

    for b in range(_NBUF):
      fb, tb, _, _, sfs, sts = slots[b]
      store_wait(fb, tb, sfs, sts)

  return k


def kernel(t5_tokens, from_table, to_table):
  batch, seq = t5_tokens.shape
  n_tokens = batch * seq
  from_dim = from_table.shape[1]
  to_dim = to_table.shape[1]

  tokens2d = t5_tokens.reshape(n_tokens // _CHUNK, _CHUNK)
  gather = _dual_gather(n_tokens, from_dim, to_dim)
  out_from, out_to = gather(tokens2d, from_table, to_table)
  return (out_from.reshape(batch, seq, from_dim),
          out_to.reshape(batch, seq, to_dim))
